# Initial kernel scaffold; baseline (speedup 1.0000x reference)
import jax
import jax.numpy as jnp
from jax.experimental import pallas as pl


def kernel(adj, feature, weight, bias):
    raise NotImplementedError("write your pallas kernel here")


# speedup vs baseline: 4.2663x; 4.2663x over previous
import math

import jax
import jax.numpy as jnp
from jax.experimental import pallas as pl
from jax.experimental.pallas import tpu as pltpu

_TM = 256
_TN = 512
_GM = 64


def _prep_body(x_ref, w_ref, a_ref, dinv_ref, y_ref):
    nslice = a_ref.shape[1] // _TN
    acc = jnp.sum(a_ref[:, 0:_TN], axis=1, keepdims=True)
    for j in range(1, nslice):
        acc = acc + jnp.sum(a_ref[:, j * _TN:(j + 1) * _TN], axis=1,
                            keepdims=True)
    dinv = jax.lax.rsqrt(jnp.maximum(acc, 1.0))
    dinv_ref[...] = dinv
    y_ref[...] = dinv * jnp.dot(x_ref[...], w_ref[...],
                                preferred_element_type=jnp.float32)


def _score_body(b_ref, y_ref, dinv_ref, a_ref, score_ref, tanh_ref):
    nslice = a_ref.shape[1] // _TN
    acc = jnp.dot(a_ref[:, 0:_TN], y_ref[0:_TN],
                  preferred_element_type=jnp.float32)
    for j in range(1, nslice):
        acc = acc + jnp.dot(a_ref[:, j * _TN:(j + 1) * _TN],
                            y_ref[j * _TN:(j + 1) * _TN],
                            preferred_element_type=jnp.float32)
    s = acc * dinv_ref[...] + b_ref[0]
    score_ref[...] = s
    tanh_ref[...] = jnp.tanh(s)


def _gather_body(perm_ref, feat_ref, ts_ref, out_ref):
    base = pl.program_id(0) * _GM
    rows = []
    for mi in range(_GM):
        r = perm_ref[base + mi]
        rows.append(feat_ref[r, 0] * ts_ref[r, 0])
    for mi in range(_GM):
        out_ref[mi] = rows[mi]


def kernel(adj, feature, weight, bias):
    n, f = feature.shape
    num_graphs = 8
    per_graph = n // num_graphs
    k = int(math.ceil(0.5 * per_graph))

    cparams = pltpu.CompilerParams(
        dimension_semantics=("parallel",),
        vmem_limit_bytes=48 * 1024 * 1024)

    dinv, y = pl.pallas_call(
        _prep_body,
        out_shape=(jax.ShapeDtypeStruct((n, 1), jnp.float32),
                   jax.ShapeDtypeStruct((n, 1), jnp.float32)),
        grid=(n // _TM,),
        in_specs=[
            pl.BlockSpec((_TM, f), lambda i: (i, 0)),
            pl.BlockSpec((f, 1), lambda i: (0, 0)),
            pl.BlockSpec((_TM, n), lambda i: (i, 0)),
        ],
        out_specs=(
            pl.BlockSpec((_TM, 1), lambda i: (i, 0)),
            pl.BlockSpec((_TM, 1), lambda i: (i, 0)),
        ),
        compiler_params=cparams,
    )(feature, weight, adj)

    score, tanh_s = pl.pallas_call(
        _score_body,
        out_shape=(jax.ShapeDtypeStruct((n, 1), jnp.float32),
                   jax.ShapeDtypeStruct((n, 1), jnp.float32)),
        grid=(n // _TM,),
        in_specs=[
            pl.BlockSpec(memory_space=pltpu.MemorySpace.SMEM),
            pl.BlockSpec((n, 1), lambda i: (0, 0)),
            pl.BlockSpec((_TM, 1), lambda i: (i, 0)),
            pl.BlockSpec((_TM, n), lambda i: (i, 0)),
        ],
        out_specs=(
            pl.BlockSpec((_TM, 1), lambda i: (i, 0)),
            pl.BlockSpec((_TM, 1), lambda i: (i, 0)),
        ),
        compiler_params=cparams,
    )(bias, y, dinv, adj)

    order = jnp.argsort(-score.reshape(num_graphs, per_graph), axis=1)
    offs = (jnp.arange(num_graphs, dtype=jnp.int32) * per_graph)[:, None]
    perm = (order[:, :k].astype(jnp.int32) + offs).reshape(-1)

    pooled = pl.pallas_call(
        _gather_body,
        out_shape=jax.ShapeDtypeStruct((perm.shape[0], f), jnp.float32),
        grid_spec=pltpu.PrefetchScalarGridSpec(
            num_scalar_prefetch=1,
            grid=(perm.shape[0] // _GM,),
            in_specs=[
                pl.BlockSpec((n, 1, f), lambda i, perm: (0, 0, 0)),
                pl.BlockSpec((n, 1, 1), lambda i, perm: (0, 0, 0)),
            ],
            out_specs=pl.BlockSpec((_GM, f), lambda i, perm: (i, 0)),
        ),
        compiler_params=pltpu.CompilerParams(
            dimension_semantics=("parallel",)),
    )(perm, feature.reshape(n, 1, f), tanh_s.reshape(n, 1, 1))

    next_batch_num_nodes = jnp.full((num_graphs,), k, dtype=jnp.int32)
    return pooled, perm, next_batch_num_nodes


# speedup vs baseline: 4.3573x; 1.0213x over previous
import math

import jax
import jax.numpy as jnp
from jax.experimental import pallas as pl
from jax.experimental.pallas import tpu as pltpu

_TM = 256
_TN = 512
_GM = 64


def _prep_body(x_ref, w_ref, a_ref, dinv_ref, y_ref):
    nslice = a_ref.shape[1] // _TN
    acc = jnp.sum(a_ref[:, 0:_TN], axis=1, keepdims=True)
    for j in range(1, nslice):
        acc = acc + jnp.sum(a_ref[:, j * _TN:(j + 1) * _TN], axis=1,
                            keepdims=True)
    dinv = jax.lax.rsqrt(jnp.maximum(acc, 1.0))
    dinv_ref[...] = dinv
    y_ref[...] = dinv * jnp.dot(x_ref[...], w_ref[...],
                                preferred_element_type=jnp.float32)


def _score_body(b_ref, y_ref, dinv_ref, a_ref, score_ref, tanh_ref):
    nslice = a_ref.shape[1] // _TN
    acc = jnp.dot(a_ref[:, 0:_TN], y_ref[0:_TN],
                  preferred_element_type=jnp.float32)
    for j in range(1, nslice):
        acc = acc + jnp.dot(a_ref[:, j * _TN:(j + 1) * _TN],
                            y_ref[j * _TN:(j + 1) * _TN],
                            preferred_element_type=jnp.float32)
    s = acc * dinv_ref[...] + b_ref[0]
    score_ref[...] = s
    tanh_ref[...] = jnp.tanh(s)


def _gather_body(perm_ref, ts_ref, feat_ref, out_ref):
    base = pl.program_id(0) * _GM
    rows = []
    for mi in range(_GM):
        r = perm_ref[base + mi]
        rows.append(feat_ref[r, 0] * ts_ref[r])
    for mi in range(_GM):
        out_ref[mi] = rows[mi]


def kernel(adj, feature, weight, bias):
    n, f = feature.shape
    num_graphs = 8
    per_graph = n // num_graphs
    k = int(math.ceil(0.5 * per_graph))

    cparams = pltpu.CompilerParams(
        dimension_semantics=("parallel",),
        vmem_limit_bytes=48 * 1024 * 1024)

    dinv, y = pl.pallas_call(
        _prep_body,
        out_shape=(jax.ShapeDtypeStruct((n, 1), jnp.float32),
                   jax.ShapeDtypeStruct((n, 1), jnp.float32)),
        grid=(n // _TM,),
        in_specs=[
            pl.BlockSpec((_TM, f), lambda i: (i, 0)),
            pl.BlockSpec((f, 1), lambda i: (0, 0)),
            pl.BlockSpec((_TM, n), lambda i: (i, 0)),
        ],
        out_specs=(
            pl.BlockSpec((_TM, 1), lambda i: (i, 0)),
            pl.BlockSpec((_TM, 1), lambda i: (i, 0)),
        ),
        compiler_params=cparams,
    )(feature, weight, adj)

    score, tanh_s = pl.pallas_call(
        _score_body,
        out_shape=(jax.ShapeDtypeStruct((n, 1), jnp.float32),
                   jax.ShapeDtypeStruct((n, 1), jnp.float32)),
        grid=(n // _TM,),
        in_specs=[
            pl.BlockSpec(memory_space=pltpu.MemorySpace.SMEM),
            pl.BlockSpec((n, 1), lambda i: (0, 0)),
            pl.BlockSpec((_TM, 1), lambda i: (i, 0)),
            pl.BlockSpec((_TM, n), lambda i: (i, 0)),
        ],
        out_specs=(
            pl.BlockSpec((_TM, 1), lambda i: (i, 0)),
            pl.BlockSpec((_TM, 1), lambda i: (i, 0)),
        ),
        compiler_params=cparams,
    )(bias, y, dinv, adj)

    order = jnp.argsort(-score.reshape(num_graphs, per_graph), axis=1)
    offs = (jnp.arange(num_graphs, dtype=jnp.int32) * per_graph)[:, None]
    perm = (order[:, :k].astype(jnp.int32) + offs).reshape(-1)

    pooled = pl.pallas_call(
        _gather_body,
        out_shape=jax.ShapeDtypeStruct((perm.shape[0], f), jnp.float32),
        grid_spec=pltpu.PrefetchScalarGridSpec(
            num_scalar_prefetch=2,
            grid=(perm.shape[0] // _GM,),
            in_specs=[
                pl.BlockSpec((n, 1, f), lambda i, perm, ts: (0, 0, 0)),
            ],
            out_specs=pl.BlockSpec((_GM, f), lambda i, perm, ts: (i, 0)),
        ),
        compiler_params=pltpu.CompilerParams(
            dimension_semantics=("parallel",)),
    )(perm, tanh_s.reshape(-1), feature.reshape(n, 1, f))

    next_batch_num_nodes = jnp.full((num_graphs,), k, dtype=jnp.int32)
    return pooled, perm, next_batch_num_nodes


# speedup vs baseline: 5.1174x; 1.1744x over previous
import math

import jax
import jax.numpy as jnp
from jax.experimental import pallas as pl
from jax.experimental.pallas import tpu as pltpu

_TM = 512
_TN = 512
_GM = 256


def _prep_body(x_ref, w_ref, a_ref, dinv_ref, y_ref):
    nslice = a_ref.shape[1] // _TN
    acc = jnp.sum(a_ref[:, 0:_TN], axis=1, keepdims=True)
    for j in range(1, nslice):
        acc = acc + jnp.sum(a_ref[:, j * _TN:(j + 1) * _TN], axis=1,
                            keepdims=True)
    dinv = jax.lax.rsqrt(jnp.maximum(acc, 1.0))
    dinv_ref[...] = dinv
    y_ref[...] = dinv * jnp.dot(x_ref[...], w_ref[...],
                                preferred_element_type=jnp.float32)


def _score_body(b_ref, y_ref, dinv_ref, a_ref, score_ref, tanh_ref):
    nslice = a_ref.shape[1] // _TN
    acc = jnp.dot(a_ref[:, 0:_TN], y_ref[0:_TN],
                  preferred_element_type=jnp.float32)
    for j in range(1, nslice):
        acc = acc + jnp.dot(a_ref[:, j * _TN:(j + 1) * _TN],
                            y_ref[j * _TN:(j + 1) * _TN],
                            preferred_element_type=jnp.float32)
    s = acc * dinv_ref[...] + b_ref[0]
    score_ref[...] = s
    tanh_ref[...] = jnp.tanh(s)


def _gather_body(perm_ref, ts_ref, feat_ref, out_ref):
    base = pl.program_id(0) * _GM
    rows = []
    for mi in range(_GM):
        r = perm_ref[base + mi]
        rows.append(feat_ref[r, 0] * ts_ref[r])
    for mi in range(_GM):
        out_ref[mi] = rows[mi]


def kernel(adj, feature, weight, bias):
    n, f = feature.shape
    num_graphs = 8
    per_graph = n // num_graphs
    k = int(math.ceil(0.5 * per_graph))

    cparams = pltpu.CompilerParams(
        dimension_semantics=("parallel",),
        vmem_limit_bytes=48 * 1024 * 1024)

    dinv, y = pl.pallas_call(
        _prep_body,
        out_shape=(jax.ShapeDtypeStruct((n, 1), jnp.float32),
                   jax.ShapeDtypeStruct((n, 1), jnp.float32)),
        grid=(n // _TM,),
        in_specs=[
            pl.BlockSpec((_TM, f), lambda i: (i, 0)),
            pl.BlockSpec((f, 1), lambda i: (0, 0)),
            pl.BlockSpec((_TM, n), lambda i: (i, 0)),
        ],
        out_specs=(
            pl.BlockSpec((_TM, 1), lambda i: (i, 0)),
            pl.BlockSpec((_TM, 1), lambda i: (i, 0)),
        ),
        compiler_params=cparams,
    )(feature, weight, adj)

    score, tanh_s = pl.pallas_call(
        _score_body,
        out_shape=(jax.ShapeDtypeStruct((n, 1), jnp.float32),
                   jax.ShapeDtypeStruct((n, 1), jnp.float32)),
        grid=(n // _TM,),
        in_specs=[
            pl.BlockSpec(memory_space=pltpu.MemorySpace.SMEM),
            pl.BlockSpec((n, 1), lambda i: (0, 0)),
            pl.BlockSpec((_TM, 1), lambda i: (i, 0)),
            pl.BlockSpec((_TM, n), lambda i: (i, 0)),
        ],
        out_specs=(
            pl.BlockSpec((_TM, 1), lambda i: (i, 0)),
            pl.BlockSpec((_TM, 1), lambda i: (i, 0)),
        ),
        compiler_params=cparams,
    )(bias, y, dinv, adj)

    order = jnp.argsort(-score.reshape(num_graphs, per_graph), axis=1)
    offs = (jnp.arange(num_graphs, dtype=jnp.int32) * per_graph)[:, None]
    perm = (order[:, :k].astype(jnp.int32) + offs).reshape(-1)

    pooled = pl.pallas_call(
        _gather_body,
        out_shape=jax.ShapeDtypeStruct((perm.shape[0], f), jnp.float32),
        grid_spec=pltpu.PrefetchScalarGridSpec(
            num_scalar_prefetch=2,
            grid=(perm.shape[0] // _GM,),
            in_specs=[
                pl.BlockSpec((n, 1, f), lambda i, perm, ts: (0, 0, 0)),
            ],
            out_specs=pl.BlockSpec((_GM, f), lambda i, perm, ts: (i, 0)),
        ),
        compiler_params=pltpu.CompilerParams(
            dimension_semantics=("parallel",)),
    )(perm, tanh_s.reshape(-1), feature.reshape(n, 1, f))

    next_batch_num_nodes = jnp.full((num_graphs,), k, dtype=jnp.int32)
    return pooled, perm, next_batch_num_nodes


# speedup vs baseline: 5.5842x; 1.0912x over previous
import math

import jax
import jax.numpy as jnp
from jax.experimental import pallas as pl
from jax.experimental.pallas import tpu as pltpu

_TN = 512
_GM = 256


def _dot_nt(a, b):
    return jax.lax.dot_general(a, b, (((1,), (1,)), ((), ())),
                               preferred_element_type=jnp.float32)


def _prep_body(wt_ref, x_ref, a_ref, dinv_ref, y_ref):
    ones = jnp.ones((1, a_ref.shape[1]), jnp.float32)
    deg = _dot_nt(ones, a_ref[...])
    dinv = jax.lax.rsqrt(jnp.maximum(deg, 1.0))
    dinv_ref[0] = dinv
    y_ref[0] = dinv * _dot_nt(wt_ref[...], x_ref[...])


def _score_body(b_ref, y_ref, dinv_ref, a_ref, neg_ref, tanh_ref):
    nslice = a_ref.shape[1] // _TN
    y = y_ref[0]
    acc = _dot_nt(y[:, 0:_TN], a_ref[:, 0:_TN])
    for j in range(1, nslice):
        acc = acc + _dot_nt(y[:, j * _TN:(j + 1) * _TN],
                            a_ref[:, j * _TN:(j + 1) * _TN])
    s = acc * dinv_ref[0] + b_ref[0]
    neg_ref[0] = -s
    tanh_ref[0] = jnp.tanh(s)


def _gather_body(perm_ref, ts_ref, feat_ref, out_ref):
    base = pl.program_id(0) * _GM
    rows = []
    for mi in range(_GM):
        r = perm_ref[base + mi]
        rows.append(feat_ref[r, 0] * ts_ref[r])
    for mi in range(_GM):
        out_ref[mi] = rows[mi]


def kernel(adj, feature, weight, bias):
    n, f = feature.shape
    num_graphs = 8
    per_graph = n // num_graphs
    k = int(math.ceil(0.5 * per_graph))
    tm = per_graph

    cparams = pltpu.CompilerParams(
        dimension_semantics=("parallel",),
        vmem_limit_bytes=48 * 1024 * 1024)

    wt = weight.reshape(1, f)

    dinv, y = pl.pallas_call(
        _prep_body,
        out_shape=(jax.ShapeDtypeStruct((1, 1, n), jnp.float32),
                   jax.ShapeDtypeStruct((1, 1, n), jnp.float32)),
        grid=(n // tm,),
        in_specs=[
            pl.BlockSpec((1, f), lambda i: (0, 0)),
            pl.BlockSpec((tm, f), lambda i: (i, 0)),
            pl.BlockSpec((tm, n), lambda i: (i, 0)),
        ],
        out_specs=(
            pl.BlockSpec((1, 1, tm), lambda i: (0, 0, i)),
            pl.BlockSpec((1, 1, tm), lambda i: (0, 0, i)),
        ),
        compiler_params=cparams,
    )(wt, feature, adj)

    neg3, tanh3 = pl.pallas_call(
        _score_body,
        out_shape=(jax.ShapeDtypeStruct((num_graphs, 1, per_graph),
                                        jnp.float32),
                   jax.ShapeDtypeStruct((num_graphs, 1, per_graph),
                                        jnp.float32)),
        grid=(n // tm,),
        in_specs=[
            pl.BlockSpec(memory_space=pltpu.MemorySpace.SMEM),
            pl.BlockSpec((1, 1, n), lambda i: (0, 0, 0)),
            pl.BlockSpec((1, 1, tm), lambda i: (0, 0, i)),
            pl.BlockSpec((tm, n), lambda i: (i, 0)),
        ],
        out_specs=(
            pl.BlockSpec((1, 1, tm), lambda i: (i, 0, 0)),
            pl.BlockSpec((1, 1, tm), lambda i: (i, 0, 0)),
        ),
        compiler_params=cparams,
    )(bias, y, dinv, adj)

    order = jnp.argsort(neg3.reshape(num_graphs, per_graph), axis=1)
    offs = (jnp.arange(num_graphs, dtype=jnp.int32) * per_graph)[:, None]
    perm = (order[:, :k].astype(jnp.int32) + offs).reshape(-1)

    pooled = pl.pallas_call(
        _gather_body,
        out_shape=jax.ShapeDtypeStruct((perm.shape[0], f), jnp.float32),
        grid_spec=pltpu.PrefetchScalarGridSpec(
            num_scalar_prefetch=2,
            grid=(perm.shape[0] // _GM,),
            in_specs=[
                pl.BlockSpec((n, 1, f), lambda i, perm, ts: (0, 0, 0)),
            ],
            out_specs=pl.BlockSpec((_GM, f), lambda i, perm, ts: (i, 0)),
        ),
        compiler_params=pltpu.CompilerParams(
            dimension_semantics=("parallel",)),
    )(perm, tanh3.reshape(-1), feature.reshape(n, 1, f))

    next_batch_num_nodes = jnp.full((num_graphs,), k, dtype=jnp.int32)
    return pooled, perm, next_batch_num_nodes


# speedup vs baseline: 7.2342x; 1.2955x over previous
import math

import jax
import jax.numpy as jnp
from jax.experimental import pallas as pl
from jax.experimental.pallas import tpu as pltpu

_TN = 512
_GM = 256


def _dot_nt(a, b):
    return jax.lax.dot_general(a, b, (((1,), (1,)), ((), ())),
                               preferred_element_type=jnp.float32)


def _dot_nn(a, b):
    return jax.lax.dot_general(a, b, (((1,), (0,)), ((), ())),
                               preferred_element_type=jnp.float32)


def _fused_body(wt_ref, x_ref, a_ref, dinv_ref, parts_ref, abuf, yloc):
    c = pl.program_id(0)
    j = pl.program_id(1)
    nstrip = pl.num_programs(1) - 1

    del c

    @pl.when(j < nstrip)
    def _phase0():
        a = a_ref[...]
        ones = jnp.ones((1, a.shape[1]), jnp.float32)
        deg = _dot_nt(ones, a)
        dinv = jax.lax.rsqrt(jnp.maximum(deg, 1.0))
        dinv_ref[0] = dinv
        yloc[j] = dinv * _dot_nt(wt_ref[...], x_ref[...])
        abuf[j] = a.astype(jnp.bfloat16)

    @pl.when(j == nstrip)
    def _phase1():
        for jj in range(nstrip):
            parts_ref[jj] = _dot_nn(yloc[jj], abuf[jj].astype(jnp.float32))


def _combine_body(b_ref, parts_ref, dinv_ref, neg_ref, tanh_ref):
    nparts, _, n = parts_ref.shape
    num_graphs = neg_ref.shape[0]
    per = neg_ref.shape[2]
    acc = parts_ref[0]
    for p in range(1, nparts):
        acc = acc + parts_ref[p]
    s = acc * dinv_ref[0] + b_ref[0]
    for g in range(num_graphs):
        sg = s[:, g * per:(g + 1) * per]
        neg_ref[g] = -sg
        tanh_ref[g] = jnp.tanh(sg)


def _gather_body(perm_ref, ts_ref, feat_ref, out_ref):
    base = pl.program_id(0) * _GM
    rows = []
    for mi in range(_GM):
        r = perm_ref[base + mi]
        rows.append(feat_ref[r, 0] * ts_ref[r])
    for mi in range(_GM):
        out_ref[mi] = rows[mi]


def kernel(adj, feature, weight, bias):
    n, f = feature.shape
    num_graphs = 8
    per_graph = n // num_graphs
    k = int(math.ceil(0.5 * per_graph))
    tm = _TN
    ncores = 2
    nstrip = n // (tm * ncores)

    wt = weight.reshape(1, f)

    def amap(c, j):
        return (c * nstrip + jnp.minimum(j, nstrip - 1), 0)

    dinv, parts = pl.pallas_call(
        _fused_body,
        out_shape=(jax.ShapeDtypeStruct((1, 1, n), jnp.float32),
                   jax.ShapeDtypeStruct((ncores * nstrip, 1, n),
                                        jnp.float32)),
        grid=(ncores, nstrip + 1),
        in_specs=[
            pl.BlockSpec((1, f), lambda c, j: (0, 0)),
            pl.BlockSpec((tm, f), amap),
            pl.BlockSpec((tm, n), amap),
        ],
        out_specs=(
            pl.BlockSpec((1, 1, tm), lambda c, j: (
                0, 0, c * nstrip + jnp.minimum(j, nstrip - 1))),
            pl.BlockSpec((nstrip, 1, n), lambda c, j: (c, 0, 0)),
        ),
        scratch_shapes=[
            pltpu.VMEM((nstrip, tm, n), jnp.bfloat16),
            pltpu.VMEM((nstrip, 1, tm), jnp.float32),
        ],
        compiler_params=pltpu.CompilerParams(
            dimension_semantics=("parallel", "arbitrary"),
            vmem_limit_bytes=60 * 1024 * 1024),
    )(wt, feature, adj)

    neg3, tanh3 = pl.pallas_call(
        _combine_body,
        out_shape=(jax.ShapeDtypeStruct((num_graphs, 1, per_graph),
                                        jnp.float32),
                   jax.ShapeDtypeStruct((num_graphs, 1, per_graph),
                                        jnp.float32)),
        in_specs=[
            pl.BlockSpec(memory_space=pltpu.MemorySpace.SMEM),
            pl.BlockSpec((ncores * nstrip, 1, n), lambda: (0, 0, 0)),
            pl.BlockSpec((1, 1, n), lambda: (0, 0, 0)),
        ],
        out_specs=(
            pl.BlockSpec((num_graphs, 1, per_graph), lambda: (0, 0, 0)),
            pl.BlockSpec((num_graphs, 1, per_graph), lambda: (0, 0, 0)),
        ),
    )(bias, parts, dinv)

    order = jnp.argsort(neg3.reshape(num_graphs, per_graph), axis=1)
    offs = (jnp.arange(num_graphs, dtype=jnp.int32) * per_graph)[:, None]
    perm = (order[:, :k].astype(jnp.int32) + offs).reshape(-1)

    pooled = pl.pallas_call(
        _gather_body,
        out_shape=jax.ShapeDtypeStruct((perm.shape[0], f), jnp.float32),
        grid_spec=pltpu.PrefetchScalarGridSpec(
            num_scalar_prefetch=2,
            grid=(perm.shape[0] // _GM,),
            in_specs=[
                pl.BlockSpec((n, 1, f), lambda i, perm, ts: (0, 0, 0)),
            ],
            out_specs=pl.BlockSpec((_GM, f), lambda i, perm, ts: (i, 0)),
        ),
        compiler_params=pltpu.CompilerParams(
            dimension_semantics=("parallel",)),
    )(perm, tanh3.reshape(-1), feature.reshape(n, 1, f))

    next_batch_num_nodes = jnp.full((num_graphs,), k, dtype=jnp.int32)
    return pooled, perm, next_batch_num_nodes


# speedup vs baseline: 7.3620x; 1.0177x over previous
import math

import jax
import jax.numpy as jnp
from jax.experimental import pallas as pl
from jax.experimental.pallas import tpu as pltpu

_TN = 512
_GM = 512


def _dot_nt(a, b):
    return jax.lax.dot_general(a, b, (((1,), (1,)), ((), ())),
                               preferred_element_type=jnp.float32)


def _dot_nn(a, b):
    return jax.lax.dot_general(a, b, (((1,), (0,)), ((), ())),
                               preferred_element_type=jnp.float32)


def _fused_body(wt_ref, x_ref, a_ref, dinv_ref, parts_ref, abuf, yloc):
    c = pl.program_id(0)
    j = pl.program_id(1)
    nstrip = pl.num_programs(1) - 1

    del c

    @pl.when(j < nstrip)
    def _phase0():
        a = a_ref[...]
        ones = jnp.ones((1, a.shape[1]), jnp.float32)
        deg = _dot_nt(ones, a)
        dinv = jax.lax.rsqrt(jnp.maximum(deg, 1.0))
        dinv_ref[0] = dinv
        yloc[j] = (dinv * _dot_nt(wt_ref[...], x_ref[...])).astype(
            jnp.bfloat16)
        abuf[j] = a.astype(jnp.bfloat16)

    @pl.when(j == nstrip)
    def _phase1():
        for jj in range(nstrip):
            parts_ref[jj] = _dot_nn(yloc[jj], abuf[jj])


def _combine_body(b_ref, parts_ref, dinv_ref, neg_ref, tanh_ref):
    nparts, _, n = parts_ref.shape
    num_graphs = neg_ref.shape[0]
    per = neg_ref.shape[2]
    acc = parts_ref[0]
    for p in range(1, nparts):
        acc = acc + parts_ref[p]
    s = acc * dinv_ref[0] + b_ref[0]
    for g in range(num_graphs):
        sg = s[:, g * per:(g + 1) * per]
        neg_ref[g] = -sg
        tanh_ref[g] = jnp.tanh(sg)


def _gather_body(perm_ref, ts_ref, feat_ref, out_ref):
    base = pl.program_id(0) * _GM
    rows = []
    for mi in range(_GM):
        r = perm_ref[base + mi]
        rows.append(feat_ref[r, 0] * ts_ref[r])
    for mi in range(_GM):
        out_ref[mi] = rows[mi]


def kernel(adj, feature, weight, bias):
    n, f = feature.shape
    num_graphs = 8
    per_graph = n // num_graphs
    k = int(math.ceil(0.5 * per_graph))
    tm = _TN
    ncores = 2
    nstrip = n // (tm * ncores)

    wt = weight.reshape(1, f)

    def amap(c, j):
        return (c * nstrip + jnp.minimum(j, nstrip - 1), 0)

    dinv, parts = pl.pallas_call(
        _fused_body,
        out_shape=(jax.ShapeDtypeStruct((1, 1, n), jnp.float32),
                   jax.ShapeDtypeStruct((ncores * nstrip, 1, n),
                                        jnp.float32)),
        grid=(ncores, nstrip + 1),
        in_specs=[
            pl.BlockSpec((1, f), lambda c, j: (0, 0)),
            pl.BlockSpec((tm, f), amap),
            pl.BlockSpec((tm, n), amap),
        ],
        out_specs=(
            pl.BlockSpec((1, 1, tm), lambda c, j: (
                0, 0, c * nstrip + jnp.minimum(j, nstrip - 1))),
            pl.BlockSpec((nstrip, 1, n), lambda c, j: (c, 0, 0)),
        ),
        scratch_shapes=[
            pltpu.VMEM((nstrip, tm, n), jnp.bfloat16),
            pltpu.VMEM((nstrip, 1, tm), jnp.bfloat16),
        ],
        compiler_params=pltpu.CompilerParams(
            dimension_semantics=("parallel", "arbitrary"),
            vmem_limit_bytes=60 * 1024 * 1024),
    )(wt, feature, adj)

    neg3, tanh3 = pl.pallas_call(
        _combine_body,
        out_shape=(jax.ShapeDtypeStruct((num_graphs, 1, per_graph),
                                        jnp.float32),
                   jax.ShapeDtypeStruct((num_graphs, 1, per_graph),
                                        jnp.float32)),
        in_specs=[
            pl.BlockSpec(memory_space=pltpu.MemorySpace.SMEM),
            pl.BlockSpec((ncores * nstrip, 1, n), lambda: (0, 0, 0)),
            pl.BlockSpec((1, 1, n), lambda: (0, 0, 0)),
        ],
        out_specs=(
            pl.BlockSpec((num_graphs, 1, per_graph), lambda: (0, 0, 0)),
            pl.BlockSpec((num_graphs, 1, per_graph), lambda: (0, 0, 0)),
        ),
    )(bias, parts, dinv)

    order = jnp.argsort(neg3.reshape(num_graphs, per_graph), axis=1)
    offs = (jnp.arange(num_graphs, dtype=jnp.int32) * per_graph)[:, None]
    perm = (order[:, :k].astype(jnp.int32) + offs).reshape(-1)

    pooled = pl.pallas_call(
        _gather_body,
        out_shape=jax.ShapeDtypeStruct((perm.shape[0], f), jnp.float32),
        grid_spec=pltpu.PrefetchScalarGridSpec(
            num_scalar_prefetch=2,
            grid=(perm.shape[0] // _GM,),
            in_specs=[
                pl.BlockSpec((n, 1, f), lambda i, perm, ts: (0, 0, 0)),
            ],
            out_specs=pl.BlockSpec((_GM, f), lambda i, perm, ts: (i, 0)),
        ),
        compiler_params=pltpu.CompilerParams(
            dimension_semantics=("parallel",)),
    )(perm, tanh3.reshape(-1), feature.reshape(n, 1, f))

    next_batch_num_nodes = jnp.full((num_graphs,), k, dtype=jnp.int32)
    return pooled, perm, next_batch_num_nodes


# speedup vs baseline: 7.8957x; 1.0725x over previous
import math

import jax
import jax.numpy as jnp
from jax.experimental import pallas as pl
from jax.experimental.pallas import tpu as pltpu

_TN = 512
_GM = 512


def _dot_nt(a, b):
    return jax.lax.dot_general(a, b, (((1,), (1,)), ((), ())),
                               preferred_element_type=jnp.float32)


def _dot_nn(a, b):
    return jax.lax.dot_general(a, b, (((1,), (0,)), ((), ())),
                               preferred_element_type=jnp.float32)


def _fused_body(wt_ref, x_ref, a_ref, dinv_ref, parts_ref):
    a = a_ref[...]
    ones = jnp.ones((1, a.shape[1]), jnp.float32)
    deg = _dot_nt(ones, a)
    dinv = jax.lax.rsqrt(jnp.maximum(deg, 1.0))
    dinv_ref[0] = dinv
    y = dinv * _dot_nt(wt_ref[...], x_ref[...])
    parts_ref[0] = _dot_nn(y, a)


def _combine_body(b_ref, parts_ref, dinv_ref, neg_ref, tanh_ref):
    nparts, _, n = parts_ref.shape
    num_graphs = neg_ref.shape[0]
    per = neg_ref.shape[2]
    acc = parts_ref[0]
    for p in range(1, nparts):
        acc = acc + parts_ref[p]
    s = acc * dinv_ref[0] + b_ref[0]
    for g in range(num_graphs):
        sg = s[:, g * per:(g + 1) * per]
        neg_ref[g] = -sg
        tanh_ref[g] = jnp.tanh(sg)


def _gather_body(perm_ref, ts_ref, feat_ref, out_ref):
    base = pl.program_id(0) * _GM
    rows = []
    for mi in range(_GM):
        r = perm_ref[base + mi]
        rows.append(feat_ref[r, 0] * ts_ref[r])
    for mi in range(_GM):
        out_ref[mi] = rows[mi]


def kernel(adj, feature, weight, bias):
    n, f = feature.shape
    num_graphs = 8
    per_graph = n // num_graphs
    k = int(math.ceil(0.5 * per_graph))
    wt = weight.reshape(1, f)

    nstrip = n // _TN
    dinv, parts = pl.pallas_call(
        _fused_body,
        out_shape=(jax.ShapeDtypeStruct((1, 1, n), jnp.float32),
                   jax.ShapeDtypeStruct((nstrip, 1, n), jnp.float32)),
        grid=(nstrip,),
        in_specs=[
            pl.BlockSpec((1, f), lambda i: (0, 0)),
            pl.BlockSpec((_TN, f), lambda i: (i, 0)),
            pl.BlockSpec((_TN, n), lambda i: (i, 0)),
        ],
        out_specs=(
            pl.BlockSpec((1, 1, _TN), lambda i: (0, 0, i)),
            pl.BlockSpec((1, 1, n), lambda i: (i, 0, 0)),
        ),
        compiler_params=pltpu.CompilerParams(
            dimension_semantics=("parallel",),
            vmem_limit_bytes=48 * 1024 * 1024),
    )(wt, feature, adj)

    neg3, tanh3 = pl.pallas_call(
        _combine_body,
        out_shape=(jax.ShapeDtypeStruct((num_graphs, 1, per_graph),
                                        jnp.float32),
                   jax.ShapeDtypeStruct((num_graphs, 1, per_graph),
                                        jnp.float32)),
        in_specs=[
            pl.BlockSpec(memory_space=pltpu.MemorySpace.SMEM),
            pl.BlockSpec((nstrip, 1, n), lambda: (0, 0, 0)),
            pl.BlockSpec((1, 1, n), lambda: (0, 0, 0)),
        ],
        out_specs=(
            pl.BlockSpec((num_graphs, 1, per_graph), lambda: (0, 0, 0)),
            pl.BlockSpec((num_graphs, 1, per_graph), lambda: (0, 0, 0)),
        ),
    )(bias, parts, dinv)

    order = jnp.argsort(neg3.reshape(num_graphs, per_graph), axis=1)
    offs = (jnp.arange(num_graphs, dtype=jnp.int32) * per_graph)[:, None]
    perm = (order[:, :k].astype(jnp.int32) + offs).reshape(-1)

    pooled = pl.pallas_call(
        _gather_body,
        out_shape=jax.ShapeDtypeStruct((perm.shape[0], f), jnp.float32),
        grid_spec=pltpu.PrefetchScalarGridSpec(
            num_scalar_prefetch=2,
            grid=(perm.shape[0] // _GM,),
            in_specs=[
                pl.BlockSpec((n, 1, f), lambda i, perm, ts: (0, 0, 0)),
            ],
            out_specs=pl.BlockSpec((_GM, f), lambda i, perm, ts: (i, 0)),
        ),
        compiler_params=pltpu.CompilerParams(
            dimension_semantics=("parallel",)),
    )(perm, tanh3.reshape(-1), feature.reshape(n, 1, f))

    next_batch_num_nodes = jnp.full((num_graphs,), k, dtype=jnp.int32)
    return pooled, perm, next_batch_num_nodes
